# Initial kernel scaffold; baseline (speedup 1.0000x reference)
import jax
import jax.numpy as jnp
from jax.experimental import pallas as pl


def kernel(x, edge_index, edge_type, W1, root1, b1, W2, root2, b2):
    raise NotImplementedError("write your pallas kernel here")


# speedup vs baseline: 9.9296x; 9.9296x over previous
import functools

import jax
import jax.numpy as jnp
from jax import lax
from jax.experimental import pallas as pl
from jax.experimental.pallas import tpu as pltpu
from jax.experimental.pallas import tpu_sc as plsc

N = 10000
E = 320000
R = 8
D_IN = 128
D_HID = 128
D_OUT = 64

NC, NS, L = 2, 16, 16
NW = NC * NS
EPW = E // NW
CH = 80
NCHUNK = EPW // CH
N_PAD = 10240
ROWS_PT = N_PAD // NS
WB = 128

_f32 = jnp.float32
_i32 = jnp.int32


def _sc_mesh():
    return plsc.VectorSubcoreMesh(
        core_axis_name="c", subcore_axis_name="s",
        num_cores=NC, num_subcores=NS)


CR = N_PAD * R // 128
CRT = CR // NS


def _count_body(dst, et, zer, out, cnt_sp, dstb, etb, rowb, onehb, cwb):
    c = lax.axis_index("c")
    s = lax.axis_index("s")
    w = c * NS + s

    @pl.when(s == 0)
    def _():
        pltpu.sync_copy(zer, cnt_sp)

    plsc.subcore_barrier()

    iota = lax.iota(_i32, L)
    ones = jnp.full((L,), 1.0, _f32)
    zeros = jnp.zeros((L,), _f32)
    for j in range(CH * 128 // L):
        flat = j * L + iota
        plsc.store_scatter(onehb, [lax.shift_right_logical(flat, 7),
                                   lax.bitwise_and(flat, 127)], zeros)

    def chunk(g, carry):
        base = w * EPW + g * CH
        pltpu.sync_copy(dst.at[pl.ds(base, CH)], dstb)
        pltpu.sync_copy(et.at[pl.ds(base, CH)], etb)
        cols = []
        for j in range(CH // L):
            sl = pl.ds(j * L, L)
            fidx = dstb[sl] * R + etb[sl]
            rowb[sl] = lax.shift_right_logical(fidx, 7)
            cols.append(lax.bitwise_and(fidx, 127))
        for j in range(CH // L):
            plsc.store_scatter(onehb, [j * L + iota, cols[j]], ones)
        pltpu.sync_copy(onehb, cnt_sp.at[rowb], add=True)
        for j in range(CH // L):
            plsc.store_scatter(onehb, [j * L + iota, cols[j]], zeros)
        return carry

    lax.fori_loop(0, NCHUNK, chunk, 0)
    plsc.subcore_barrier()

    pltpu.sync_copy(cnt_sp.at[pl.ds(s * CRT, CRT)], cwb)
    pltpu.sync_copy(cwb, out.at[c, pl.ds(s * CRT, CRT)])


_count_kernel = functools.partial(
    pl.kernel,
    out_type=jax.ShapeDtypeStruct((NC, CR, 128), _f32),
    mesh=_sc_mesh(),
    compiler_params=pltpu.CompilerParams(needs_layout_passes=False),
    scratch_types=[
        pltpu.VMEM_SHARED((CR, 128), _f32),
        pltpu.VMEM((CH,), _i32),
        pltpu.VMEM((CH,), _i32),
        pltpu.VMEM((CH,), _i32),
        pltpu.VMEM((CH, 128), _f32),
        pltpu.VMEM((CRT, 128), _f32),
    ],
)(_count_body)


def _scale_body(dst, et, inv, scl, invv, dstb, etb, sclb):
    c = lax.axis_index("c")
    s = lax.axis_index("s")
    w = c * NS + s
    pltpu.sync_copy(inv, invv)
    iota = lax.iota(_i32, L)

    def chunk(g, carry):
        base = w * EPW + g * CH
        pltpu.sync_copy(dst.at[pl.ds(base, CH)], dstb)
        pltpu.sync_copy(et.at[pl.ds(base, CH)], etb)
        for j5 in range(CH // L):
            sl5 = pl.ds(j5 * L, L)
            sclb[sl5] = plsc.load_gather(invv, [dstb[sl5] * R + etb[sl5]])
        pltpu.sync_copy(sclb, scl.at[pl.ds(base, CH)])
        return carry

    lax.fori_loop(0, NCHUNK, chunk, 0)


_scale_kernel = functools.partial(
    pl.kernel,
    out_type=jax.ShapeDtypeStruct((E,), _f32),
    mesh=_sc_mesh(),
    compiler_params=pltpu.CompilerParams(needs_layout_passes=False),
    scratch_types=[
        pltpu.VMEM((N * R,), _f32),
        pltpu.VMEM((CH,), _i32),
        pltpu.VMEM((CH,), _i32),
        pltpu.VMEM((CH,), _f32),
    ],
)(_scale_body)


def _agg_body(H, src, dst, et, xwf, scl, zer, out,
              agg_sp, srcb, dstb, etb, gib, sclb, rows, wb):
    c = lax.axis_index("c")
    s = lax.axis_index("s")
    w = c * NS + s

    pltpu.sync_copy(zer.at[pl.ds(s * ROWS_PT, ROWS_PT)],
                    agg_sp.at[pl.ds(s * ROWS_PT, ROWS_PT)])
    plsc.subcore_barrier()

    iota = lax.iota(_i32, L)

    def chunk(g, carry):
        base = w * EPW + g * CH
        pltpu.sync_copy(src.at[pl.ds(base, CH)], srcb)
        pltpu.sync_copy(dst.at[pl.ds(base, CH)], dstb)
        pltpu.sync_copy(et.at[pl.ds(base, CH)], etb)
        for j in range(CH // L):
            sl = pl.ds(j * L, L)
            gib[sl] = etb[sl] * N + srcb[sl]
        pltpu.sync_copy(scl.at[pl.ds(base, CH)], sclb)
        pltpu.sync_copy(xwf.at[gib], rows)
        for j5 in range(CH // L):
            sl5 = pl.ds(j5 * L, L)
            scv = sclb[sl5]
            for t in range(L):
                i = j5 * L + t
                sc = scv[t]
                for j in range(H // L):
                    sl = pl.ds(j * L, L)
                    rows[i, sl] = rows[i, sl] * sc
        pltpu.sync_copy(rows, agg_sp.at[dstb], add=True)
        return carry

    lax.fori_loop(0, NCHUNK, chunk, 0)
    plsc.subcore_barrier()

    for k in range(ROWS_PT // WB):
        r0 = s * ROWS_PT + k * WB
        pltpu.sync_copy(agg_sp.at[pl.ds(r0, WB)], wb)
        pltpu.sync_copy(wb, out.at[c, pl.ds(r0, WB)])


def _make_agg_kernel(H):
    return functools.partial(
        pl.kernel,
        out_type=jax.ShapeDtypeStruct((NC, N_PAD, H), _f32),
        mesh=_sc_mesh(),
        compiler_params=pltpu.CompilerParams(needs_layout_passes=False),
        scratch_types=[
            pltpu.VMEM_SHARED((N_PAD, H), _f32),
            pltpu.VMEM((CH,), _i32),
            pltpu.VMEM((CH,), _i32),
            pltpu.VMEM((CH,), _i32),
            pltpu.VMEM((CH,), _i32),
            pltpu.VMEM((CH,), _f32),
            pltpu.VMEM((CH, H), _f32),
            pltpu.VMEM((WB, H), _f32),
        ],
    )(functools.partial(_agg_body, H))


_agg_kernel_hid = _make_agg_kernel(D_HID)


def _inv_body(cnt_ref, inv_ref):
    c = cnt_ref[...]
    inv_ref[...] = 1.0 / jnp.maximum(c[0] + c[1], 1.0)


def _inv_call(cnt):
    return pl.pallas_call(
        _inv_body,
        out_shape=jax.ShapeDtypeStruct((CR, 128), _f32),
    )(cnt)


_BN = 400
_NB = N // _BN


def _mm_body(x_ref, w_ref, o_ref):
    o_ref[...] = jnp.dot(x_ref[...], w_ref[0],
                         preferred_element_type=_f32)


def _mm_call(x, wall, H):
    nrel = wall.shape[0]
    return pl.pallas_call(
        _mm_body,
        grid=(_NB, nrel),
        in_specs=[
            pl.BlockSpec((_BN, x.shape[1]), lambda i, r: (i, 0)),
            pl.BlockSpec((1, x.shape[1], H), lambda i, r: (r, 0, 0)),
        ],
        out_specs=pl.BlockSpec((_BN, H), lambda i, r: (r * _NB + i, 0)),
        out_shape=jax.ShapeDtypeStruct((nrel * N, H), _f32),
    )(x, wall)


def _comb_body(relu, xr_ref, agg_ref, b_ref, o_ref):
    y = xr_ref[...] + agg_ref[0] + agg_ref[1] + b_ref[...]
    if relu:
        y = jnp.maximum(y, 0.0)
    o_ref[...] = y


def _comb_call(xwf, agg, b, H, relu):
    return pl.pallas_call(
        functools.partial(_comb_body, relu),
        grid=(_NB,),
        in_specs=[
            pl.BlockSpec((_BN, H), lambda i: (R * _NB + i, 0)),
            pl.BlockSpec((NC, _BN, H), lambda i: (0, i, 0)),
            pl.BlockSpec((1, H), lambda i: (0, 0)),
        ],
        out_specs=pl.BlockSpec((_BN, H), lambda i: (i, 0)),
        out_shape=jax.ShapeDtypeStruct((N, H), _f32),
    )(xwf, agg, b)


def _final_body(xr_ref, agg_ref, b_ref, o_ref):
    i = pl.program_id(0)

    @pl.when(i == 0)
    def _():
        o_ref[...] = jnp.zeros_like(o_ref)

    blk = xr_ref[...][:, :D_OUT] + agg_ref[0] + agg_ref[1]
    o_ref[...] += jnp.sum(blk, axis=0, keepdims=True)

    @pl.when(i == _NB - 1)
    def _():
        t = o_ref[...] / N + b_ref[...]
        m = jnp.max(t, axis=1, keepdims=True)
        ls = t - m
        o_ref[...] = ls - jnp.log(jnp.sum(jnp.exp(ls), axis=1,
                                          keepdims=True))


def _final_call(xwf, agg, b):
    return pl.pallas_call(
        _final_body,
        grid=(_NB,),
        in_specs=[
            pl.BlockSpec((_BN, D_HID), lambda i: (R * _NB + i, 0)),
            pl.BlockSpec((NC, _BN, D_OUT), lambda i: (0, i, 0)),
            pl.BlockSpec((1, D_OUT), lambda i: (0, 0)),
        ],
        out_specs=pl.BlockSpec((1, D_OUT), lambda i: (0, 0)),
        out_shape=jax.ShapeDtypeStruct((1, D_OUT), _f32),
    )(xwf, agg, b)


def kernel(x, edge_index, edge_type, W1, root1, b1, W2, root2, b2):
    zer_nr = jnp.zeros((CR, 128), _f32)
    zer_hid = jnp.zeros((N_PAD, D_HID), _f32)

    src = edge_index[0]
    dst = edge_index[1]
    cnt = _count_kernel(dst, edge_type, zer_nr)
    inv = _inv_call(cnt).reshape(N_PAD * R)[:N * R]
    scl = _scale_kernel(dst, edge_type, inv)

    wall1 = jnp.concatenate([W1, root1[None]], axis=0)
    xwf1 = _mm_call(x, wall1, D_HID)
    agg1 = _agg_kernel_hid(src, dst, edge_type, xwf1, scl, zer_hid)[:, :N]
    h = _comb_call(xwf1, agg1, b1.reshape(1, D_HID), D_HID, True)

    wall2 = jnp.concatenate([W2, root2[None]], axis=0)
    wall2p = jnp.pad(wall2, ((0, 0), (0, 0), (0, D_HID - D_OUT)))
    xwf2 = _mm_call(h, wall2p, D_HID)
    agg2 = _agg_kernel_hid(src, dst, edge_type, xwf2, scl, zer_hid)
    agg2 = agg2[:, :N, :D_OUT]
    return _final_call(xwf2, agg2, b2.reshape(1, D_OUT))


# speedup vs baseline: 16.6116x; 1.6729x over previous
import functools

import jax
import jax.numpy as jnp
from jax import lax
from jax.experimental import pallas as pl
from jax.experimental.pallas import tpu as pltpu
from jax.experimental.pallas import tpu_sc as plsc

N = 10000
E = 320000
R = 8
D_IN = 128
D_HID = 128
D_OUT = 64

NC, NS, L = 2, 16, 16
NW = NC * NS
EPW = E // NW
CH = 80
NCHUNK = EPW // CH
N_PAD = 10240
ROWS_PT = N_PAD // NS

_f32 = jnp.float32
_i32 = jnp.int32


def _sc_mesh():
    return plsc.VectorSubcoreMesh(
        core_axis_name="c", subcore_axis_name="s",
        num_cores=NC, num_subcores=NS)


CR = N_PAD * R // 128
CRT = CR // NS


def _count_body(dst, et, zer, out, cnt_sp, dstb, etb, rowb, onehb, cwb):
    c = lax.axis_index("c")
    s = lax.axis_index("s")
    w = c * NS + s

    @pl.when(s == 0)
    def _():
        pltpu.sync_copy(zer, cnt_sp)

    plsc.subcore_barrier()

    iota = lax.iota(_i32, L)
    ones = jnp.full((L,), 1.0, _f32)
    zeros = jnp.zeros((L,), _f32)
    for j in range(CH * 128 // L):
        flat = j * L + iota
        plsc.store_scatter(onehb, [lax.shift_right_logical(flat, 7),
                                   lax.bitwise_and(flat, 127)], zeros)

    def chunk(g, carry):
        base = w * EPW + g * CH
        pltpu.sync_copy(dst.at[pl.ds(base, CH)], dstb)
        pltpu.sync_copy(et.at[pl.ds(base, CH)], etb)
        cols = []
        for j in range(CH // L):
            sl = pl.ds(j * L, L)
            fidx = dstb[sl] * R + etb[sl]
            rowb[sl] = lax.shift_right_logical(fidx, 7)
            cols.append(lax.bitwise_and(fidx, 127))
        for j in range(CH // L):
            plsc.store_scatter(onehb, [j * L + iota, cols[j]], ones)
        pltpu.sync_copy(onehb, cnt_sp.at[rowb], add=True)
        for j in range(CH // L):
            plsc.store_scatter(onehb, [j * L + iota, cols[j]], zeros)
        return carry

    lax.fori_loop(0, NCHUNK, chunk, 0)
    plsc.subcore_barrier()

    pltpu.sync_copy(cnt_sp.at[pl.ds(s * CRT, CRT)], cwb)
    pltpu.sync_copy(cwb, out.at[c, pl.ds(s * CRT, CRT)])


_count_kernel = functools.partial(
    pl.kernel,
    out_type=jax.ShapeDtypeStruct((NC, CR, 128), _f32),
    mesh=_sc_mesh(),
    compiler_params=pltpu.CompilerParams(needs_layout_passes=False),
    scratch_types=[
        pltpu.VMEM_SHARED((CR, 128), _f32),
        pltpu.VMEM((CH,), _i32),
        pltpu.VMEM((CH,), _i32),
        pltpu.VMEM((CH,), _i32),
        pltpu.VMEM((CH, 128), _f32),
        pltpu.VMEM((CRT, 128), _f32),
    ],
)(_count_body)


def _scale_body(dst, et, inv, scl, invv, dstb, etb, sclb):
    c = lax.axis_index("c")
    s = lax.axis_index("s")
    w = c * NS + s
    pltpu.sync_copy(inv, invv)
    iota = lax.iota(_i32, L)

    def chunk(g, carry):
        base = w * EPW + g * CH
        pltpu.sync_copy(dst.at[pl.ds(base, CH)], dstb)
        pltpu.sync_copy(et.at[pl.ds(base, CH)], etb)
        for j5 in range(CH // L):
            sl5 = pl.ds(j5 * L, L)
            sclb[sl5] = plsc.load_gather(invv, [dstb[sl5] * R + etb[sl5]])
        pltpu.sync_copy(sclb, scl.at[pl.ds(base, CH)])
        return carry

    lax.fori_loop(0, NCHUNK, chunk, 0)


_scale_kernel = functools.partial(
    pl.kernel,
    out_type=jax.ShapeDtypeStruct((E,), _f32),
    mesh=_sc_mesh(),
    compiler_params=pltpu.CompilerParams(needs_layout_passes=False),
    scratch_types=[
        pltpu.VMEM((N * R,), _f32),
        pltpu.VMEM((CH,), _i32),
        pltpu.VMEM((CH,), _i32),
        pltpu.VMEM((CH,), _f32),
    ],
)(_scale_body)


STG = 400
NSTG = EPW // STG
WB = 64


def _agg_body(H, src, dst, et, xwf, scl, zer, out,
              agg_sp, gibf, stg_s, stg_t,
              dstb0, dstb1, sclb0, sclb1, rows0, rows1, wb,
              esem0, esem1, gsem0, gsem1, ssem0, ssem1):
    c = lax.axis_index("c")
    s = lax.axis_index("s")
    w = c * NS + s
    ebase = w * EPW

    pltpu.sync_copy(zer.at[pl.ds(s * ROWS_PT, ROWS_PT)],
                    agg_sp.at[pl.ds(s * ROWS_PT, ROWS_PT)])

    def stage(k, carry):
        b0 = ebase + k * STG
        pltpu.sync_copy(src.at[pl.ds(b0, STG)], stg_s)
        pltpu.sync_copy(et.at[pl.ds(b0, STG)], stg_t)
        for j in range(STG // L):
            sl = pl.ds(j * L, L)
            gibf[pl.ds(k * STG + j * L, L)] = stg_t[sl] * N + stg_s[sl]
        return carry

    lax.fori_loop(0, NSTG, stage, 0)
    plsc.subcore_barrier()

    bufs = ((dstb0, sclb0, rows0, esem0, gsem0, ssem0),
            (dstb1, sclb1, rows1, esem1, gsem1, ssem1))

    def issue_a(g, bf):
        dstb, sclb, rows, esem, gsem, _ = bf
        base = ebase + g * CH
        pltpu.async_copy(dst.at[pl.ds(base, CH)], dstb, esem)
        pltpu.async_copy(scl.at[pl.ds(base, CH)], sclb, esem)
        pltpu.async_copy(xwf.at[gibf.at[pl.ds(g * CH, CH)]], rows, gsem)

    def wait_scatter(bf):
        _, _, rows, _, _, ssem = bf
        pltpu.make_async_copy(rows, agg_sp.at[pl.ds(0, CH)], ssem).wait()

    def proc(g, cur, nxt):
        @pl.when(g < NCHUNK)
        def _():
            dstb, sclb, rows, esem, gsem, ssem = cur

            @pl.when(g + 1 < NCHUNK)
            def _():
                @pl.when(g >= 1)
                def _():
                    wait_scatter(nxt)
                issue_a(g + 1, nxt)

            pltpu.make_async_copy(dst.at[pl.ds(0, CH)], dstb, esem).wait()
            pltpu.make_async_copy(scl.at[pl.ds(0, CH)], sclb, esem).wait()
            pltpu.make_async_copy(xwf.at[gibf.at[pl.ds(0, CH)]], rows,
                                  gsem).wait()
            for j5 in range(CH // L):
                scv = sclb[pl.ds(j5 * L, L)]
                for t in range(L):
                    i = j5 * L + t
                    sc = scv[t]
                    for j in range(H // L):
                        sl = pl.ds(j * L, L)
                        rows[i, sl] = rows[i, sl] * sc
            pltpu.async_copy(rows, agg_sp.at[dstb], ssem, add=True)

    issue_a(0, bufs[0])

    def pair(gg, carry):
        proc(2 * gg, bufs[0], bufs[1])
        proc(2 * gg + 1, bufs[1], bufs[0])
        return carry

    lax.fori_loop(0, (NCHUNK + 2) // 2, pair, 0)
    wait_scatter(bufs[0])
    wait_scatter(bufs[1])
    plsc.subcore_barrier()

    for k in range(ROWS_PT // WB):
        r0 = s * ROWS_PT + k * WB
        pltpu.sync_copy(agg_sp.at[pl.ds(r0, WB)], wb)
        pltpu.sync_copy(wb, out.at[c, pl.ds(r0, WB)])


def _make_agg_kernel(H):
    return functools.partial(
        pl.kernel,
        out_type=jax.ShapeDtypeStruct((NC, N_PAD, H), _f32),
        mesh=_sc_mesh(),
        compiler_params=pltpu.CompilerParams(needs_layout_passes=False),
        scratch_types=[
            pltpu.VMEM_SHARED((N_PAD, H), _f32),
            pltpu.VMEM((EPW,), _i32),
            pltpu.VMEM((STG,), _i32),
            pltpu.VMEM((STG,), _i32),
            pltpu.VMEM((CH,), _i32),
            pltpu.VMEM((CH,), _i32),
            pltpu.VMEM((CH,), _f32),
            pltpu.VMEM((CH,), _f32),
            pltpu.VMEM((CH, H), _f32),
            pltpu.VMEM((CH, H), _f32),
            pltpu.VMEM((WB, H), _f32),
            pltpu.SemaphoreType.DMA,
            pltpu.SemaphoreType.DMA,
            pltpu.SemaphoreType.DMA,
            pltpu.SemaphoreType.DMA,
            pltpu.SemaphoreType.DMA,
            pltpu.SemaphoreType.DMA,
        ],
    )(functools.partial(_agg_body, H))


_agg_kernel_hid = _make_agg_kernel(D_HID)


def _inv_body(cnt_ref, inv_ref):
    c = cnt_ref[...]
    inv_ref[...] = 1.0 / jnp.maximum(c[0] + c[1], 1.0)


def _inv_call(cnt):
    return pl.pallas_call(
        _inv_body,
        out_shape=jax.ShapeDtypeStruct((CR, 128), _f32),
    )(cnt)


_BN = 400
_NB = N // _BN


def _mm_body(x_ref, w_ref, o_ref):
    o_ref[...] = jnp.dot(x_ref[...], w_ref[0],
                         preferred_element_type=_f32)


def _mm_call(x, wall, H):
    nrel = wall.shape[0]
    return pl.pallas_call(
        _mm_body,
        grid=(_NB, nrel),
        in_specs=[
            pl.BlockSpec((_BN, x.shape[1]), lambda i, r: (i, 0)),
            pl.BlockSpec((1, x.shape[1], H), lambda i, r: (r, 0, 0)),
        ],
        out_specs=pl.BlockSpec((_BN, H), lambda i, r: (r * _NB + i, 0)),
        out_shape=jax.ShapeDtypeStruct((nrel * N, H), _f32),
    )(x, wall)


def _comb_body(relu, xr_ref, agg_ref, b_ref, o_ref):
    y = xr_ref[...] + agg_ref[0] + agg_ref[1] + b_ref[...]
    if relu:
        y = jnp.maximum(y, 0.0)
    o_ref[...] = y


def _comb_call(xwf, agg, b, H, relu):
    return pl.pallas_call(
        functools.partial(_comb_body, relu),
        grid=(_NB,),
        in_specs=[
            pl.BlockSpec((_BN, H), lambda i: (R * _NB + i, 0)),
            pl.BlockSpec((NC, _BN, H), lambda i: (0, i, 0)),
            pl.BlockSpec((1, H), lambda i: (0, 0)),
        ],
        out_specs=pl.BlockSpec((_BN, H), lambda i: (i, 0)),
        out_shape=jax.ShapeDtypeStruct((N, H), _f32),
    )(xwf, agg, b)


def _final_body(xr_ref, agg_ref, b_ref, o_ref):
    i = pl.program_id(0)

    @pl.when(i == 0)
    def _():
        o_ref[...] = jnp.zeros_like(o_ref)

    blk = xr_ref[...][:, :D_OUT] + agg_ref[0] + agg_ref[1]
    o_ref[...] += jnp.sum(blk, axis=0, keepdims=True)

    @pl.when(i == _NB - 1)
    def _():
        t = o_ref[...] / N + b_ref[...]
        m = jnp.max(t, axis=1, keepdims=True)
        ls = t - m
        o_ref[...] = ls - jnp.log(jnp.sum(jnp.exp(ls), axis=1,
                                          keepdims=True))


def _final_call(xwf, agg, b):
    return pl.pallas_call(
        _final_body,
        grid=(_NB,),
        in_specs=[
            pl.BlockSpec((_BN, D_HID), lambda i: (R * _NB + i, 0)),
            pl.BlockSpec((NC, _BN, D_OUT), lambda i: (0, i, 0)),
            pl.BlockSpec((1, D_OUT), lambda i: (0, 0)),
        ],
        out_specs=pl.BlockSpec((1, D_OUT), lambda i: (0, 0)),
        out_shape=jax.ShapeDtypeStruct((1, D_OUT), _f32),
    )(xwf, agg, b)


def kernel(x, edge_index, edge_type, W1, root1, b1, W2, root2, b2):
    zer_nr = jnp.zeros((CR, 128), _f32)
    zer_hid = jnp.zeros((N_PAD, D_HID), _f32)

    src = edge_index[0]
    dst = edge_index[1]
    cnt = _count_kernel(dst, edge_type, zer_nr)
    inv = _inv_call(cnt).reshape(N_PAD * R)[:N * R]
    scl = _scale_kernel(dst, edge_type, inv)

    wall1 = jnp.concatenate([W1, root1[None]], axis=0)
    xwf1 = _mm_call(x, wall1, D_HID)
    agg1 = _agg_kernel_hid(src, dst, edge_type, xwf1, scl, zer_hid)[:, :N]
    h = _comb_call(xwf1, agg1, b1.reshape(1, D_HID), D_HID, True)

    wall2 = jnp.concatenate([W2, root2[None]], axis=0)
    wall2p = jnp.pad(wall2, ((0, 0), (0, 0), (0, D_HID - D_OUT)))
    xwf2 = _mm_call(h, wall2p, D_HID)
    agg2 = _agg_kernel_hid(src, dst, edge_type, xwf2, scl, zer_hid)
    agg2 = agg2[:, :N, :D_OUT]
    return _final_call(xwf2, agg2, b2.reshape(1, D_OUT))


# speedup vs baseline: 20.7920x; 1.2517x over previous
import functools

import jax
import jax.numpy as jnp
from jax import lax
from jax.experimental import pallas as pl
from jax.experimental.pallas import tpu as pltpu
from jax.experimental.pallas import tpu_sc as plsc

N = 10000
E = 320000
R = 8
D_IN = 128
D_HID = 128
D_OUT = 64

NC, NS, L = 2, 16, 16
NW = NC * NS
EPW = E // NW
CH = 80
NCHUNK = EPW // CH
N_PAD = 10240
ROWS_PT = N_PAD // NS

_f32 = jnp.float32
_i32 = jnp.int32


def _sc_mesh():
    return plsc.VectorSubcoreMesh(
        core_axis_name="c", subcore_axis_name="s",
        num_cores=NC, num_subcores=NS)


EPC = E // NS
NCHC = EPC // CH
NR = N_PAD * R


def _prep_body(dst, et, zer, scl,
               cnt_sp, cntv,
               dstb0, dstb1, etb0, etb1, fidx0, fidx1, onesb,
               sclb0, sclb1,
               esem0, esem1, ssem0, ssem1, osem0, osem1):
    c = lax.axis_index("c")
    s = lax.axis_index("s")
    w = c * NS + s

    pltpu.sync_copy(zer.at[pl.ds(s * (NR // NS), NR // NS)],
                    cnt_sp.at[pl.ds(s * (NR // NS), NR // NS)])

    iota = lax.iota(_i32, L)
    ones = jnp.full((L,), 1.0, _f32)
    for j in range(CH // L):
        onesb[pl.ds(j * L, L)] = ones
    plsc.subcore_barrier()

    cbufs = ((dstb0, etb0, fidx0, esem0, ssem0),
             (dstb1, etb1, fidx1, esem1, ssem1))
    cbase = s * EPC

    def cload(g, bf):
        base = cbase + g * CH
        pltpu.async_copy(dst.at[pl.ds(base, CH)], bf[0], bf[3])
        pltpu.async_copy(et.at[pl.ds(base, CH)], bf[1], bf[3])

    def cproc(g, bf):
        dstb, etb, fidx, esem, ssem = bf
        pltpu.make_async_copy(dst.at[pl.ds(0, CH)], dstb, esem).wait()
        pltpu.make_async_copy(et.at[pl.ds(0, CH)], etb, esem).wait()

        @pl.when(g >= 2)
        def _():
            pltpu.make_async_copy(onesb, cnt_sp.at[pl.ds(0, CH)],
                                  ssem).wait()

        for j in range(CH // L):
            sl = pl.ds(j * L, L)
            fidx[sl] = dstb[sl] * R + etb[sl]
        pltpu.async_copy(onesb, cnt_sp.at[fidx], ssem, add=True)

        @pl.when(g + 2 < NCHC)
        def _():
            cload(g + 2, bf)

    cload(0, cbufs[0])
    cload(1, cbufs[1])

    def cpair(gg, carry):
        cproc(2 * gg, cbufs[0])
        cproc(2 * gg + 1, cbufs[1])
        return carry

    lax.fori_loop(0, NCHC // 2, cpair, 0)
    for bf in cbufs:
        pltpu.make_async_copy(onesb, cnt_sp.at[pl.ds(0, CH)], bf[4]).wait()
    plsc.subcore_barrier()

    pltpu.sync_copy(cnt_sp, cntv)

    sbufs = ((dstb0, etb0, fidx0, sclb0, esem0, osem0),
             (dstb1, etb1, fidx1, sclb1, esem1, osem1))
    ebase = w * EPW

    def sload(g, bf):
        base = ebase + g * CH
        pltpu.async_copy(dst.at[pl.ds(base, CH)], bf[0], bf[4])
        pltpu.async_copy(et.at[pl.ds(base, CH)], bf[1], bf[4])

    def sproc(g, bf):
        @pl.when(g < NCHUNK)
        def _():
            dstb, etb, fidx, sclb, esem, osem = bf
            pltpu.make_async_copy(dst.at[pl.ds(0, CH)], dstb, esem).wait()
            pltpu.make_async_copy(et.at[pl.ds(0, CH)], etb, esem).wait()

            @pl.when(g >= 2)
            def _():
                pltpu.make_async_copy(
                    sclb, scl.at[pl.ds(0, CH)], osem).wait()

            for j in range(CH // L):
                sl = pl.ds(j * L, L)
                cv = plsc.load_gather(cntv, [dstb[sl] * R + etb[sl]])
                sclb[sl] = 1.0 / jnp.maximum(cv, 1.0)
            pltpu.async_copy(sclb, scl.at[pl.ds(ebase + g * CH, CH)], osem)

            @pl.when(g + 2 < NCHUNK)
            def _():
                sload(g + 2, bf)

    sload(0, sbufs[0])
    sload(1, sbufs[1])

    def spair(gg, carry):
        sproc(2 * gg, sbufs[0])
        sproc(2 * gg + 1, sbufs[1])
        return carry

    lax.fori_loop(0, (NCHUNK + 1) // 2, spair, 0)
    for bf in sbufs:
        pltpu.make_async_copy(bf[3], scl.at[pl.ds(0, CH)], bf[5]).wait()


_prep_kernel = functools.partial(
    pl.kernel,
    out_type=jax.ShapeDtypeStruct((E,), _f32),
    mesh=_sc_mesh(),
    compiler_params=pltpu.CompilerParams(needs_layout_passes=False),
    scratch_types=[
        pltpu.VMEM_SHARED((NR,), _f32),
        pltpu.VMEM((NR,), _f32),
        pltpu.VMEM((CH,), _i32),
        pltpu.VMEM((CH,), _i32),
        pltpu.VMEM((CH,), _i32),
        pltpu.VMEM((CH,), _i32),
        pltpu.VMEM((CH,), _i32),
        pltpu.VMEM((CH,), _i32),
        pltpu.VMEM((CH,), _f32),
        pltpu.VMEM((CH,), _f32),
        pltpu.VMEM((CH,), _f32),
        pltpu.SemaphoreType.DMA,
        pltpu.SemaphoreType.DMA,
        pltpu.SemaphoreType.DMA,
        pltpu.SemaphoreType.DMA,
        pltpu.SemaphoreType.DMA,
        pltpu.SemaphoreType.DMA,
    ],
)(_prep_body)


STG = 400
NSTG = EPW // STG
WB = 64


def _agg_body(H, src, dst, et, xwf, scl, zer, out,
              agg_sp, gibf, stg_s, stg_t,
              dstb0, dstb1, sclb0, sclb1, rows0, rows1, wb,
              esem0, esem1, gsem0, gsem1, ssem0, ssem1):
    c = lax.axis_index("c")
    s = lax.axis_index("s")
    w = c * NS + s
    ebase = w * EPW

    pltpu.sync_copy(zer.at[pl.ds(s * ROWS_PT, ROWS_PT)],
                    agg_sp.at[pl.ds(s * ROWS_PT, ROWS_PT)])

    def stage(k, carry):
        b0 = ebase + k * STG
        pltpu.sync_copy(src.at[pl.ds(b0, STG)], stg_s)
        pltpu.sync_copy(et.at[pl.ds(b0, STG)], stg_t)
        for j in range(STG // L):
            sl = pl.ds(j * L, L)
            gibf[pl.ds(k * STG + j * L, L)] = stg_t[sl] * N + stg_s[sl]
        return carry

    lax.fori_loop(0, NSTG, stage, 0)
    plsc.subcore_barrier()

    bufs = ((dstb0, sclb0, rows0, esem0, gsem0, ssem0),
            (dstb1, sclb1, rows1, esem1, gsem1, ssem1))

    def issue_a(g, bf):
        dstb, sclb, rows, esem, gsem, _ = bf
        base = ebase + g * CH
        pltpu.async_copy(dst.at[pl.ds(base, CH)], dstb, esem)
        pltpu.async_copy(scl.at[pl.ds(base, CH)], sclb, esem)
        pltpu.async_copy(xwf.at[gibf.at[pl.ds(g * CH, CH)]], rows, gsem)

    def wait_scatter(bf):
        _, _, rows, _, _, ssem = bf
        pltpu.make_async_copy(rows, agg_sp.at[pl.ds(0, CH)], ssem).wait()

    def proc(g, cur, nxt):
        @pl.when(g < NCHUNK)
        def _():
            dstb, sclb, rows, esem, gsem, ssem = cur

            @pl.when(g + 1 < NCHUNK)
            def _():
                @pl.when(g >= 1)
                def _():
                    wait_scatter(nxt)
                issue_a(g + 1, nxt)

            pltpu.make_async_copy(dst.at[pl.ds(0, CH)], dstb, esem).wait()
            pltpu.make_async_copy(scl.at[pl.ds(0, CH)], sclb, esem).wait()
            pltpu.make_async_copy(xwf.at[gibf.at[pl.ds(0, CH)]], rows,
                                  gsem).wait()
            for j5 in range(CH // L):
                scv = sclb[pl.ds(j5 * L, L)]
                for t in range(L):
                    i = j5 * L + t
                    sc = scv[t]
                    for j in range(H // L):
                        sl = pl.ds(j * L, L)
                        rows[i, sl] = rows[i, sl] * sc
            pltpu.async_copy(rows, agg_sp.at[dstb], ssem, add=True)

    issue_a(0, bufs[0])

    def pair(gg, carry):
        proc(2 * gg, bufs[0], bufs[1])
        proc(2 * gg + 1, bufs[1], bufs[0])
        return carry

    lax.fori_loop(0, (NCHUNK + 2) // 2, pair, 0)
    wait_scatter(bufs[0])
    wait_scatter(bufs[1])
    plsc.subcore_barrier()

    for k in range(ROWS_PT // WB):
        r0 = s * ROWS_PT + k * WB
        pltpu.sync_copy(agg_sp.at[pl.ds(r0, WB)], wb)
        pltpu.sync_copy(wb, out.at[c, pl.ds(r0, WB)])


def _make_agg_kernel(H):
    return functools.partial(
        pl.kernel,
        out_type=jax.ShapeDtypeStruct((NC, N_PAD, H), _f32),
        mesh=_sc_mesh(),
        compiler_params=pltpu.CompilerParams(needs_layout_passes=False),
        scratch_types=[
            pltpu.VMEM_SHARED((N_PAD, H), _f32),
            pltpu.VMEM((EPW,), _i32),
            pltpu.VMEM((STG,), _i32),
            pltpu.VMEM((STG,), _i32),
            pltpu.VMEM((CH,), _i32),
            pltpu.VMEM((CH,), _i32),
            pltpu.VMEM((CH,), _f32),
            pltpu.VMEM((CH,), _f32),
            pltpu.VMEM((CH, H), _f32),
            pltpu.VMEM((CH, H), _f32),
            pltpu.VMEM((WB, H), _f32),
            pltpu.SemaphoreType.DMA,
            pltpu.SemaphoreType.DMA,
            pltpu.SemaphoreType.DMA,
            pltpu.SemaphoreType.DMA,
            pltpu.SemaphoreType.DMA,
            pltpu.SemaphoreType.DMA,
        ],
    )(functools.partial(_agg_body, H))


_agg_kernel_hid = _make_agg_kernel(D_HID)


_BN = 400
_NB = N // _BN


def _mm_body(x_ref, w_ref, o_ref):
    o_ref[...] = jnp.dot(x_ref[...], w_ref[0],
                         preferred_element_type=_f32)


def _mm_call(x, wall, H):
    nrel = wall.shape[0]
    return pl.pallas_call(
        _mm_body,
        grid=(_NB, nrel),
        in_specs=[
            pl.BlockSpec((_BN, x.shape[1]), lambda i, r: (i, 0)),
            pl.BlockSpec((1, x.shape[1], H), lambda i, r: (r, 0, 0)),
        ],
        out_specs=pl.BlockSpec((_BN, H), lambda i, r: (r * _NB + i, 0)),
        out_shape=jax.ShapeDtypeStruct((nrel * N, H), _f32),
    )(x, wall)


def _comb_body(relu, xr_ref, agg_ref, b_ref, o_ref):
    y = xr_ref[...] + agg_ref[0] + agg_ref[1] + b_ref[...]
    if relu:
        y = jnp.maximum(y, 0.0)
    o_ref[...] = y


def _comb_call(xwf, agg, b, H, relu):
    return pl.pallas_call(
        functools.partial(_comb_body, relu),
        grid=(_NB,),
        in_specs=[
            pl.BlockSpec((_BN, H), lambda i: (R * _NB + i, 0)),
            pl.BlockSpec((NC, _BN, H), lambda i: (0, i, 0)),
            pl.BlockSpec((1, H), lambda i: (0, 0)),
        ],
        out_specs=pl.BlockSpec((_BN, H), lambda i: (i, 0)),
        out_shape=jax.ShapeDtypeStruct((N, H), _f32),
    )(xwf, agg, b)


def _final_body(xr_ref, agg_ref, b_ref, o_ref):
    i = pl.program_id(0)

    @pl.when(i == 0)
    def _():
        o_ref[...] = jnp.zeros_like(o_ref)

    blk = xr_ref[...][:, :D_OUT] + agg_ref[0] + agg_ref[1]
    o_ref[...] += jnp.sum(blk, axis=0, keepdims=True)

    @pl.when(i == _NB - 1)
    def _():
        t = o_ref[...] / N + b_ref[...]
        m = jnp.max(t, axis=1, keepdims=True)
        ls = t - m
        o_ref[...] = ls - jnp.log(jnp.sum(jnp.exp(ls), axis=1,
                                          keepdims=True))


def _final_call(xwf, agg, b):
    return pl.pallas_call(
        _final_body,
        grid=(_NB,),
        in_specs=[
            pl.BlockSpec((_BN, D_HID), lambda i: (R * _NB + i, 0)),
            pl.BlockSpec((NC, _BN, D_OUT), lambda i: (0, i, 0)),
            pl.BlockSpec((1, D_OUT), lambda i: (0, 0)),
        ],
        out_specs=pl.BlockSpec((1, D_OUT), lambda i: (0, 0)),
        out_shape=jax.ShapeDtypeStruct((1, D_OUT), _f32),
    )(xwf, agg, b)


def kernel(x, edge_index, edge_type, W1, root1, b1, W2, root2, b2):
    zer_nr = jnp.zeros((NR,), _f32)
    zer_hid = jnp.zeros((N_PAD, D_HID), _f32)

    src = edge_index[0]
    dst = edge_index[1]
    scl = _prep_kernel(dst, edge_type, zer_nr)

    wall1 = jnp.concatenate([W1, root1[None]], axis=0)
    xwf1 = _mm_call(x, wall1, D_HID)
    agg1 = _agg_kernel_hid(src, dst, edge_type, xwf1, scl, zer_hid)[:, :N]
    h = _comb_call(xwf1, agg1, b1.reshape(1, D_HID), D_HID, True)

    wall2 = jnp.concatenate([W2, root2[None]], axis=0)
    wall2p = jnp.pad(wall2, ((0, 0), (0, 0), (0, D_HID - D_OUT)))
    xwf2 = _mm_call(h, wall2p, D_HID)
    agg2 = _agg_kernel_hid(src, dst, edge_type, xwf2, scl, zer_hid)
    agg2 = agg2[:, :N, :D_OUT]
    return _final_call(xwf2, agg2, b2.reshape(1, D_OUT))


# speedup vs baseline: 21.2277x; 1.0210x over previous
import functools

import jax
import jax.numpy as jnp
from jax import lax
from jax.experimental import pallas as pl
from jax.experimental.pallas import tpu as pltpu
from jax.experimental.pallas import tpu_sc as plsc

N = 10000
E = 320000
R = 8
D_IN = 128
D_HID = 128
D_OUT = 64

NC, NS, L = 2, 16, 16
NW = NC * NS
EPW = E // NW
CH = 80
NCHUNK = EPW // CH
N_PAD = 10240
ROWS_PT = N_PAD // NS

_f32 = jnp.float32
_i32 = jnp.int32


def _sc_mesh():
    return plsc.VectorSubcoreMesh(
        core_axis_name="c", subcore_axis_name="s",
        num_cores=NC, num_subcores=NS)


EPC = E // NS
NCHC = EPC // CH
NR = N_PAD * R


def _prep_body(dst, et, zer, scl,
               cnt_sp, cntv,
               dstb0, dstb1, etb0, etb1, fidx0, fidx1, onesb,
               sclb0, sclb1,
               esem0, esem1, ssem0, ssem1, osem0, osem1):
    c = lax.axis_index("c")
    s = lax.axis_index("s")
    w = c * NS + s

    pltpu.sync_copy(zer.at[pl.ds(s * (NR // NS), NR // NS)],
                    cnt_sp.at[pl.ds(s * (NR // NS), NR // NS)])

    iota = lax.iota(_i32, L)
    ones = jnp.full((L,), 1.0, _f32)
    for j in range(CH // L):
        onesb[pl.ds(j * L, L)] = ones
    plsc.subcore_barrier()

    cbufs = ((dstb0, etb0, fidx0, esem0, ssem0),
             (dstb1, etb1, fidx1, esem1, ssem1))
    cbase = s * EPC

    def cload(g, bf):
        base = cbase + g * CH
        pltpu.async_copy(dst.at[pl.ds(base, CH)], bf[0], bf[3])
        pltpu.async_copy(et.at[pl.ds(base, CH)], bf[1], bf[3])

    def cproc(g, bf):
        dstb, etb, fidx, esem, ssem = bf
        pltpu.make_async_copy(dst.at[pl.ds(0, CH)], dstb, esem).wait()
        pltpu.make_async_copy(et.at[pl.ds(0, CH)], etb, esem).wait()

        @pl.when(g >= 2)
        def _():
            pltpu.make_async_copy(onesb, cnt_sp.at[pl.ds(0, CH)],
                                  ssem).wait()

        for j in range(CH // L):
            sl = pl.ds(j * L, L)
            fidx[sl] = dstb[sl] * R + etb[sl]
        pltpu.async_copy(onesb, cnt_sp.at[fidx], ssem, add=True)

        @pl.when(g + 2 < NCHC)
        def _():
            cload(g + 2, bf)

    cload(0, cbufs[0])
    cload(1, cbufs[1])

    def cpair(gg, carry):
        cproc(2 * gg, cbufs[0])
        cproc(2 * gg + 1, cbufs[1])
        return carry

    lax.fori_loop(0, NCHC // 2, cpair, 0)
    for bf in cbufs:
        pltpu.make_async_copy(onesb, cnt_sp.at[pl.ds(0, CH)], bf[4]).wait()
    plsc.subcore_barrier()

    pltpu.sync_copy(cnt_sp, cntv)

    sbufs = ((dstb0, etb0, fidx0, sclb0, esem0, osem0),
             (dstb1, etb1, fidx1, sclb1, esem1, osem1))
    ebase = w * EPW

    def sload(g, bf):
        base = ebase + g * CH
        pltpu.async_copy(dst.at[pl.ds(base, CH)], bf[0], bf[4])
        pltpu.async_copy(et.at[pl.ds(base, CH)], bf[1], bf[4])

    def sproc(g, bf):
        @pl.when(g < NCHUNK)
        def _():
            dstb, etb, fidx, sclb, esem, osem = bf
            pltpu.make_async_copy(dst.at[pl.ds(0, CH)], dstb, esem).wait()
            pltpu.make_async_copy(et.at[pl.ds(0, CH)], etb, esem).wait()

            @pl.when(g >= 2)
            def _():
                pltpu.make_async_copy(
                    sclb, scl.at[pl.ds(0, CH)], osem).wait()

            for j in range(CH // L):
                sl = pl.ds(j * L, L)
                cv = plsc.load_gather(cntv, [dstb[sl] * R + etb[sl]])
                sclb[sl] = 1.0 / jnp.maximum(cv, 1.0)
            pltpu.async_copy(sclb, scl.at[pl.ds(ebase + g * CH, CH)], osem)

            @pl.when(g + 2 < NCHUNK)
            def _():
                sload(g + 2, bf)

    sload(0, sbufs[0])
    sload(1, sbufs[1])

    def spair(gg, carry):
        sproc(2 * gg, sbufs[0])
        sproc(2 * gg + 1, sbufs[1])
        return carry

    lax.fori_loop(0, (NCHUNK + 1) // 2, spair, 0)
    for bf in sbufs:
        pltpu.make_async_copy(bf[3], scl.at[pl.ds(0, CH)], bf[5]).wait()


_prep_kernel = functools.partial(
    pl.kernel,
    out_type=jax.ShapeDtypeStruct((E,), _f32),
    mesh=_sc_mesh(),
    compiler_params=pltpu.CompilerParams(needs_layout_passes=False),
    scratch_types=[
        pltpu.VMEM_SHARED((NR,), _f32),
        pltpu.VMEM((NR,), _f32),
        pltpu.VMEM((CH,), _i32),
        pltpu.VMEM((CH,), _i32),
        pltpu.VMEM((CH,), _i32),
        pltpu.VMEM((CH,), _i32),
        pltpu.VMEM((CH,), _i32),
        pltpu.VMEM((CH,), _i32),
        pltpu.VMEM((CH,), _f32),
        pltpu.VMEM((CH,), _f32),
        pltpu.VMEM((CH,), _f32),
        pltpu.SemaphoreType.DMA,
        pltpu.SemaphoreType.DMA,
        pltpu.SemaphoreType.DMA,
        pltpu.SemaphoreType.DMA,
        pltpu.SemaphoreType.DMA,
        pltpu.SemaphoreType.DMA,
    ],
)(_prep_body)


STG = 400
NSTG = EPW // STG
WB = 64


def _agg_body(H, src, dst, et, xwf, scl, zer, out,
              agg_sp, gibf, stg_s, stg_t,
              dstb0, dstb1, sclb0, sclb1, rows0, rows1, wb,
              esem0, esem1, gsem0, gsem1, ssem0, ssem1):
    c = lax.axis_index("c")
    s = lax.axis_index("s")
    w = c * NS + s
    ebase = w * EPW

    pltpu.sync_copy(zer.at[pl.ds(s * ROWS_PT, ROWS_PT)],
                    agg_sp.at[pl.ds(s * ROWS_PT, ROWS_PT)])

    def stage(k, carry):
        b0 = ebase + k * STG
        pltpu.sync_copy(src.at[pl.ds(b0, STG)], stg_s)
        pltpu.sync_copy(et.at[pl.ds(b0, STG)], stg_t)
        for j in range(STG // L):
            sl = pl.ds(j * L, L)
            gibf[pl.ds(k * STG + j * L, L)] = stg_t[sl] * N + stg_s[sl]
        return carry

    lax.fori_loop(0, NSTG, stage, 0)
    plsc.subcore_barrier()

    bufs = ((dstb0, sclb0, rows0, esem0, gsem0, ssem0),
            (dstb1, sclb1, rows1, esem1, gsem1, ssem1))

    def issue_a(g, bf):
        dstb, sclb, rows, esem, gsem, _ = bf
        base = ebase + g * CH
        pltpu.async_copy(dst.at[pl.ds(base, CH)], dstb, esem)
        pltpu.async_copy(scl.at[pl.ds(base, CH)], sclb, esem)
        pltpu.async_copy(xwf.at[gibf.at[pl.ds(g * CH, CH)]], rows, gsem)

    def wait_scatter(bf):
        _, _, rows, _, _, ssem = bf
        pltpu.make_async_copy(rows, agg_sp.at[pl.ds(0, CH)], ssem).wait()

    def proc(g, cur, nxt):
        @pl.when(g < NCHUNK)
        def _():
            dstb, sclb, rows, esem, gsem, ssem = cur

            @pl.when(g + 1 < NCHUNK)
            def _():
                @pl.when(g >= 1)
                def _():
                    wait_scatter(nxt)
                issue_a(g + 1, nxt)

            pltpu.make_async_copy(dst.at[pl.ds(0, CH)], dstb, esem).wait()
            pltpu.make_async_copy(scl.at[pl.ds(0, CH)], sclb, esem).wait()
            pltpu.make_async_copy(xwf.at[gibf.at[pl.ds(0, CH)]], rows,
                                  gsem).wait()
            for j5 in range(CH // L):
                scv = sclb[pl.ds(j5 * L, L)]
                for t in range(L):
                    i = j5 * L + t
                    sc = scv[t]
                    for j in range(H // L):
                        sl = pl.ds(j * L, L)
                        rows[i, sl] = rows[i, sl] * sc
            pltpu.async_copy(rows, agg_sp.at[dstb], ssem, add=True)

    issue_a(0, bufs[0])

    def pair(gg, carry):
        proc(2 * gg, bufs[0], bufs[1])
        proc(2 * gg + 1, bufs[1], bufs[0])
        return carry

    lax.fori_loop(0, (NCHUNK + 2) // 2, pair, 0)
    wait_scatter(bufs[0])
    wait_scatter(bufs[1])
    plsc.subcore_barrier()

    for k in range(ROWS_PT // WB):
        r0 = s * ROWS_PT + k * WB
        pltpu.sync_copy(agg_sp.at[pl.ds(r0, WB)], wb)
        pltpu.sync_copy(wb, out.at[c, pl.ds(r0, WB)])


def _make_agg_kernel(H):
    return functools.partial(
        pl.kernel,
        out_type=jax.ShapeDtypeStruct((NC, N_PAD, H), _f32),
        mesh=_sc_mesh(),
        compiler_params=pltpu.CompilerParams(needs_layout_passes=False),
        scratch_types=[
            pltpu.VMEM_SHARED((N_PAD, H), _f32),
            pltpu.VMEM((EPW,), _i32),
            pltpu.VMEM((STG,), _i32),
            pltpu.VMEM((STG,), _i32),
            pltpu.VMEM((CH,), _i32),
            pltpu.VMEM((CH,), _i32),
            pltpu.VMEM((CH,), _f32),
            pltpu.VMEM((CH,), _f32),
            pltpu.VMEM((CH, H), _f32),
            pltpu.VMEM((CH, H), _f32),
            pltpu.VMEM((WB, H), _f32),
            pltpu.SemaphoreType.DMA,
            pltpu.SemaphoreType.DMA,
            pltpu.SemaphoreType.DMA,
            pltpu.SemaphoreType.DMA,
            pltpu.SemaphoreType.DMA,
            pltpu.SemaphoreType.DMA,
        ],
    )(functools.partial(_agg_body, H))


_agg_kernel_hid = _make_agg_kernel(D_HID)


_BN = 400
_NB = N // _BN


def _mm_body(x_ref, w_ref, o_ref):
    o_ref[...] = jnp.dot(x_ref[...], w_ref[0],
                         preferred_element_type=_f32)


def _mm_call(x, wall, H):
    nrel = wall.shape[0]
    return pl.pallas_call(
        _mm_body,
        grid=(_NB, nrel),
        in_specs=[
            pl.BlockSpec((_BN, x.shape[1]), lambda i, r: (i, 0)),
            pl.BlockSpec((1, x.shape[1], H), lambda i, r: (r, 0, 0)),
        ],
        out_specs=pl.BlockSpec((_BN, H), lambda i, r: (r * _NB + i, 0)),
        out_shape=jax.ShapeDtypeStruct((nrel * N, H), _f32),
    )(x, wall)


def _comb_body(relu, xr_ref, agg_ref, b_ref, o_ref):
    y = xr_ref[...] + agg_ref[0] + agg_ref[1] + b_ref[...]
    if relu:
        y = jnp.maximum(y, 0.0)
    o_ref[...] = y


def _comb_call(xwf, agg, b, H, relu):
    return pl.pallas_call(
        functools.partial(_comb_body, relu),
        grid=(_NB,),
        in_specs=[
            pl.BlockSpec((_BN, H), lambda i: (R * _NB + i, 0)),
            pl.BlockSpec((NC, _BN, H), lambda i: (0, i, 0)),
            pl.BlockSpec((1, H), lambda i: (0, 0)),
        ],
        out_specs=pl.BlockSpec((_BN, H), lambda i: (i, 0)),
        out_shape=jax.ShapeDtypeStruct((N, H), _f32),
    )(xwf, agg, b)


def _final_body(xr_ref, agg_ref, b_ref, o_ref):
    i = pl.program_id(0)

    @pl.when(i == 0)
    def _():
        o_ref[...] = jnp.zeros_like(o_ref)

    blk = (xr_ref[...][:, :D_OUT] + agg_ref[0][:, :D_OUT]
           + agg_ref[1][:, :D_OUT])
    o_ref[...] += jnp.sum(blk, axis=0, keepdims=True)

    @pl.when(i == _NB - 1)
    def _():
        t = o_ref[...] / N + b_ref[...]
        m = jnp.max(t, axis=1, keepdims=True)
        ls = t - m
        o_ref[...] = ls - jnp.log(jnp.sum(jnp.exp(ls), axis=1,
                                          keepdims=True))


def _final_call(xwf, agg, b):
    return pl.pallas_call(
        _final_body,
        grid=(_NB,),
        in_specs=[
            pl.BlockSpec((_BN, D_HID), lambda i: (R * _NB + i, 0)),
            pl.BlockSpec((NC, _BN, D_HID), lambda i: (0, i, 0)),
            pl.BlockSpec((1, D_OUT), lambda i: (0, 0)),
        ],
        out_specs=pl.BlockSpec((1, D_OUT), lambda i: (0, 0)),
        out_shape=jax.ShapeDtypeStruct((1, D_OUT), _f32),
    )(xwf, agg, b)


def kernel(x, edge_index, edge_type, W1, root1, b1, W2, root2, b2):
    zer_nr = jnp.zeros((NR,), _f32)
    zer_hid = jnp.zeros((N_PAD, D_HID), _f32)

    src = edge_index[0]
    dst = edge_index[1]
    scl = _prep_kernel(dst, edge_type, zer_nr)

    wall1 = jnp.concatenate([W1, root1[None]], axis=0)
    xwf1 = _mm_call(x, wall1, D_HID)
    agg1 = _agg_kernel_hid(src, dst, edge_type, xwf1, scl, zer_hid)
    h = _comb_call(xwf1, agg1, b1.reshape(1, D_HID), D_HID, True)

    wall2 = jnp.concatenate([W2, root2[None]], axis=0)
    wall2p = jnp.pad(wall2, ((0, 0), (0, 0), (0, D_HID - D_OUT)))
    xwf2 = _mm_call(h, wall2p, D_HID)
    agg2 = _agg_kernel_hid(src, dst, edge_type, xwf2, scl, zer_hid)
    return _final_call(xwf2, agg2, b2.reshape(1, D_OUT))


# speedup vs baseline: 21.6170x; 1.0183x over previous
import functools

import jax
import jax.numpy as jnp
from jax import lax
from jax.experimental import pallas as pl
from jax.experimental.pallas import tpu as pltpu
from jax.experimental.pallas import tpu_sc as plsc

N = 10000
E = 320000
R = 8
D_IN = 128
D_HID = 128
D_OUT = 64

NC, NS, L = 2, 16, 16
NW = NC * NS
EPW = E // NW
CH = 80
NCHUNK = EPW // CH
N_PAD = 10240
ROWS_PT = N_PAD // NS

_f32 = jnp.float32
_i32 = jnp.int32


def _sc_mesh():
    return plsc.VectorSubcoreMesh(
        core_axis_name="c", subcore_axis_name="s",
        num_cores=NC, num_subcores=NS)


EPC = E // NS
NCHC = EPC // CH
NR = N_PAD * R


def _prep_body(dst, et, zer, scl,
               cnt_sp, cntv,
               dstb0, dstb1, etb0, etb1, fidx0, fidx1, onesb,
               sclb0, sclb1,
               esem0, esem1, ssem0, ssem1, osem0, osem1):
    c = lax.axis_index("c")
    s = lax.axis_index("s")
    w = c * NS + s

    pltpu.sync_copy(zer.at[pl.ds(s * (NR // NS), NR // NS)],
                    cnt_sp.at[pl.ds(s * (NR // NS), NR // NS)])

    iota = lax.iota(_i32, L)
    ones = jnp.full((L,), 1.0, _f32)
    for j in range(CH // L):
        onesb[pl.ds(j * L, L)] = ones
    plsc.subcore_barrier()

    cbufs = ((dstb0, etb0, fidx0, esem0, ssem0),
             (dstb1, etb1, fidx1, esem1, ssem1))
    cbase = s * EPC

    def cload(g, bf):
        base = cbase + g * CH
        pltpu.async_copy(dst.at[pl.ds(base, CH)], bf[0], bf[3])
        pltpu.async_copy(et.at[pl.ds(base, CH)], bf[1], bf[3])

    def cproc(g, bf):
        dstb, etb, fidx, esem, ssem = bf
        pltpu.make_async_copy(dst.at[pl.ds(0, CH)], dstb, esem).wait()
        pltpu.make_async_copy(et.at[pl.ds(0, CH)], etb, esem).wait()

        @pl.when(g >= 2)
        def _():
            pltpu.make_async_copy(onesb, cnt_sp.at[pl.ds(0, CH)],
                                  ssem).wait()

        for j in range(CH // L):
            sl = pl.ds(j * L, L)
            fidx[sl] = dstb[sl] * R + etb[sl]
        pltpu.async_copy(onesb, cnt_sp.at[fidx], ssem, add=True)

        @pl.when(g + 2 < NCHC)
        def _():
            cload(g + 2, bf)

    cload(0, cbufs[0])
    cload(1, cbufs[1])

    def cpair(gg, carry):
        cproc(2 * gg, cbufs[0])
        cproc(2 * gg + 1, cbufs[1])
        return carry

    lax.fori_loop(0, NCHC // 2, cpair, 0)
    for bf in cbufs:
        pltpu.make_async_copy(onesb, cnt_sp.at[pl.ds(0, CH)], bf[4]).wait()
    plsc.subcore_barrier()

    pltpu.sync_copy(cnt_sp, cntv)

    sbufs = ((dstb0, etb0, fidx0, sclb0, esem0, osem0),
             (dstb1, etb1, fidx1, sclb1, esem1, osem1))
    ebase = w * EPW

    def sload(g, bf):
        base = ebase + g * CH
        pltpu.async_copy(dst.at[pl.ds(base, CH)], bf[0], bf[4])
        pltpu.async_copy(et.at[pl.ds(base, CH)], bf[1], bf[4])

    def sproc(g, bf):
        @pl.when(g < NCHUNK)
        def _():
            dstb, etb, fidx, sclb, esem, osem = bf
            pltpu.make_async_copy(dst.at[pl.ds(0, CH)], dstb, esem).wait()
            pltpu.make_async_copy(et.at[pl.ds(0, CH)], etb, esem).wait()

            @pl.when(g >= 2)
            def _():
                pltpu.make_async_copy(
                    sclb, scl.at[pl.ds(0, CH)], osem).wait()

            for j in range(CH // L):
                sl = pl.ds(j * L, L)
                cv = plsc.load_gather(cntv, [dstb[sl] * R + etb[sl]])
                sclb[sl] = 1.0 / jnp.maximum(cv, 1.0)
            pltpu.async_copy(sclb, scl.at[pl.ds(ebase + g * CH, CH)], osem)

            @pl.when(g + 2 < NCHUNK)
            def _():
                sload(g + 2, bf)

    sload(0, sbufs[0])
    sload(1, sbufs[1])

    def spair(gg, carry):
        sproc(2 * gg, sbufs[0])
        sproc(2 * gg + 1, sbufs[1])
        return carry

    lax.fori_loop(0, (NCHUNK + 1) // 2, spair, 0)
    for bf in sbufs:
        pltpu.make_async_copy(bf[3], scl.at[pl.ds(0, CH)], bf[5]).wait()


_prep_kernel = functools.partial(
    pl.kernel,
    out_type=jax.ShapeDtypeStruct((E,), _f32),
    mesh=_sc_mesh(),
    compiler_params=pltpu.CompilerParams(needs_layout_passes=False),
    scratch_types=[
        pltpu.VMEM_SHARED((NR,), _f32),
        pltpu.VMEM((NR,), _f32),
        pltpu.VMEM((CH,), _i32),
        pltpu.VMEM((CH,), _i32),
        pltpu.VMEM((CH,), _i32),
        pltpu.VMEM((CH,), _i32),
        pltpu.VMEM((CH,), _i32),
        pltpu.VMEM((CH,), _i32),
        pltpu.VMEM((CH,), _f32),
        pltpu.VMEM((CH,), _f32),
        pltpu.VMEM((CH,), _f32),
        pltpu.SemaphoreType.DMA,
        pltpu.SemaphoreType.DMA,
        pltpu.SemaphoreType.DMA,
        pltpu.SemaphoreType.DMA,
        pltpu.SemaphoreType.DMA,
        pltpu.SemaphoreType.DMA,
    ],
)(_prep_body)


STG = 400
NSTG = EPW // STG
WB = 64


def _agg_body(H, src, dst, et, xwf, scl, zer, out,
              agg_sp, gibf, stg_s, stg_t,
              dstb0, dstb1, sclb0, sclb1, rows0, rows1, wb,
              esem0, esem1, gsem0, gsem1, ssem0, ssem1):
    c = lax.axis_index("c")
    s = lax.axis_index("s")
    w = c * NS + s
    ebase = w * EPW

    pltpu.sync_copy(zer.at[pl.ds(s * ROWS_PT, ROWS_PT)],
                    agg_sp.at[pl.ds(s * ROWS_PT, ROWS_PT)])

    def stage(k, carry):
        b0 = ebase + k * STG
        pltpu.sync_copy(src.at[pl.ds(b0, STG)], stg_s)
        pltpu.sync_copy(et.at[pl.ds(b0, STG)], stg_t)
        for j in range(STG // L):
            sl = pl.ds(j * L, L)
            gibf[pl.ds(k * STG + j * L, L)] = stg_t[sl] * N + stg_s[sl]
        return carry

    lax.fori_loop(0, NSTG, stage, 0)
    plsc.subcore_barrier()

    bufs = ((dstb0, sclb0, rows0, esem0, gsem0, ssem0),
            (dstb1, sclb1, rows1, esem1, gsem1, ssem1))

    def issue_a(g, bf):
        dstb, sclb, rows, esem, gsem, _ = bf
        base = ebase + g * CH
        pltpu.async_copy(dst.at[pl.ds(base, CH)], dstb, esem)
        pltpu.async_copy(scl.at[pl.ds(base, CH)], sclb, esem)
        pltpu.async_copy(xwf.at[gibf.at[pl.ds(g * CH, CH)]], rows, gsem)

    def wait_scatter(bf):
        _, _, rows, _, _, ssem = bf
        pltpu.make_async_copy(rows, agg_sp.at[pl.ds(0, CH)], ssem).wait()

    def proc(g, cur, nxt):
        @pl.when(g < NCHUNK)
        def _():
            dstb, sclb, rows, esem, gsem, ssem = cur

            @pl.when(g + 1 < NCHUNK)
            def _():
                @pl.when(g >= 1)
                def _():
                    wait_scatter(nxt)
                issue_a(g + 1, nxt)

            pltpu.make_async_copy(dst.at[pl.ds(0, CH)], dstb, esem).wait()
            pltpu.make_async_copy(scl.at[pl.ds(0, CH)], sclb, esem).wait()
            pltpu.make_async_copy(xwf.at[gibf.at[pl.ds(0, CH)]], rows,
                                  gsem).wait()
            for j5 in range(CH // L):
                scv = sclb[pl.ds(j5 * L, L)]
                for t in range(L):
                    i = j5 * L + t
                    sc = scv[t]
                    for j in range(H // L):
                        sl = pl.ds(j * L, L)
                        rows[i, sl] = rows[i, sl] * sc
            pltpu.async_copy(rows, agg_sp.at[dstb], ssem, add=True)

    issue_a(0, bufs[0])

    def pair(gg, carry):
        proc(2 * gg, bufs[0], bufs[1])
        proc(2 * gg + 1, bufs[1], bufs[0])
        return carry

    lax.fori_loop(0, (NCHUNK + 2) // 2, pair, 0)
    wait_scatter(bufs[0])
    wait_scatter(bufs[1])
    plsc.subcore_barrier()

    for k in range(ROWS_PT // WB):
        r0 = s * ROWS_PT + k * WB
        pltpu.sync_copy(agg_sp.at[pl.ds(r0, WB)], wb)
        pltpu.sync_copy(wb, out.at[c, pl.ds(r0, WB)])


def _make_agg_kernel(H):
    return functools.partial(
        pl.kernel,
        out_type=jax.ShapeDtypeStruct((NC, N_PAD, H), _f32),
        mesh=_sc_mesh(),
        compiler_params=pltpu.CompilerParams(needs_layout_passes=False),
        scratch_types=[
            pltpu.VMEM_SHARED((N_PAD, H), _f32),
            pltpu.VMEM((EPW,), _i32),
            pltpu.VMEM((STG,), _i32),
            pltpu.VMEM((STG,), _i32),
            pltpu.VMEM((CH,), _i32),
            pltpu.VMEM((CH,), _i32),
            pltpu.VMEM((CH,), _f32),
            pltpu.VMEM((CH,), _f32),
            pltpu.VMEM((CH, H), _f32),
            pltpu.VMEM((CH, H), _f32),
            pltpu.VMEM((WB, H), _f32),
            pltpu.SemaphoreType.DMA,
            pltpu.SemaphoreType.DMA,
            pltpu.SemaphoreType.DMA,
            pltpu.SemaphoreType.DMA,
            pltpu.SemaphoreType.DMA,
            pltpu.SemaphoreType.DMA,
        ],
    )(functools.partial(_agg_body, H))


_agg_kernel_hid = _make_agg_kernel(D_HID)


_BN = 400
_NB = N // _BN


def _mm_body(x_ref, w_ref, o_ref):
    o_ref[...] = jnp.dot(x_ref[...], w_ref[0],
                         preferred_element_type=_f32)


def _mm_call(x, wall, H):
    nrel = wall.shape[0]
    return pl.pallas_call(
        _mm_body,
        grid=(_NB, nrel),
        in_specs=[
            pl.BlockSpec((_BN, x.shape[1]), lambda i, r: (i, 0)),
            pl.BlockSpec((1, x.shape[1], H), lambda i, r: (r, 0, 0)),
        ],
        out_specs=pl.BlockSpec((_BN, H), lambda i, r: (r * _NB + i, 0)),
        out_shape=jax.ShapeDtypeStruct((nrel * N, H), _f32),
    )(x, wall)


def _mm2_body(xr_ref, agg_ref, b_ref, w_ref, o_ref):
    h = jnp.maximum(xr_ref[...] + agg_ref[0] + agg_ref[1] + b_ref[...],
                    0.0)
    o_ref[...] = jnp.dot(h, w_ref[0], preferred_element_type=_f32)


def _mm2_fused(xwf1, agg1, b, wall2):
    return pl.pallas_call(
        _mm2_body,
        grid=(_NB, 9),
        in_specs=[
            pl.BlockSpec((_BN, D_HID), lambda i, r: (R * _NB + i, 0)),
            pl.BlockSpec((NC, _BN, D_HID), lambda i, r: (0, i, 0)),
            pl.BlockSpec((1, D_HID), lambda i, r: (0, 0)),
            pl.BlockSpec((1, D_HID, D_HID), lambda i, r: (r, 0, 0)),
        ],
        out_specs=pl.BlockSpec((_BN, D_HID), lambda i, r: (r * _NB + i, 0)),
        out_shape=jax.ShapeDtypeStruct((9 * N, D_HID), _f32),
    )(xwf1, agg1, b, wall2)


def _comb_body(relu, xr_ref, agg_ref, b_ref, o_ref):
    y = xr_ref[...] + agg_ref[0] + agg_ref[1] + b_ref[...]
    if relu:
        y = jnp.maximum(y, 0.0)
    o_ref[...] = y


def _comb_call(xwf, agg, b, H, relu):
    return pl.pallas_call(
        functools.partial(_comb_body, relu),
        grid=(_NB,),
        in_specs=[
            pl.BlockSpec((_BN, H), lambda i: (R * _NB + i, 0)),
            pl.BlockSpec((NC, _BN, H), lambda i: (0, i, 0)),
            pl.BlockSpec((1, H), lambda i: (0, 0)),
        ],
        out_specs=pl.BlockSpec((_BN, H), lambda i: (i, 0)),
        out_shape=jax.ShapeDtypeStruct((N, H), _f32),
    )(xwf, agg, b)


def _final_body(xr_ref, agg_ref, b_ref, o_ref):
    i = pl.program_id(0)

    @pl.when(i == 0)
    def _():
        o_ref[...] = jnp.zeros_like(o_ref)

    blk = (xr_ref[...][:, :D_OUT] + agg_ref[0][:, :D_OUT]
           + agg_ref[1][:, :D_OUT])
    o_ref[...] += jnp.sum(blk, axis=0, keepdims=True)

    @pl.when(i == _NB - 1)
    def _():
        t = o_ref[...] / N + b_ref[...]
        m = jnp.max(t, axis=1, keepdims=True)
        ls = t - m
        o_ref[...] = ls - jnp.log(jnp.sum(jnp.exp(ls), axis=1,
                                          keepdims=True))


def _final_call(xwf, agg, b):
    return pl.pallas_call(
        _final_body,
        grid=(_NB,),
        in_specs=[
            pl.BlockSpec((_BN, D_HID), lambda i: (R * _NB + i, 0)),
            pl.BlockSpec((NC, _BN, D_HID), lambda i: (0, i, 0)),
            pl.BlockSpec((1, D_OUT), lambda i: (0, 0)),
        ],
        out_specs=pl.BlockSpec((1, D_OUT), lambda i: (0, 0)),
        out_shape=jax.ShapeDtypeStruct((1, D_OUT), _f32),
    )(xwf, agg, b)


def kernel(x, edge_index, edge_type, W1, root1, b1, W2, root2, b2):
    zer_nr = jnp.zeros((NR,), _f32)
    zer_hid = jnp.zeros((N_PAD, D_HID), _f32)

    src = edge_index[0]
    dst = edge_index[1]
    scl = _prep_kernel(dst, edge_type, zer_nr)

    wall1 = jnp.concatenate([W1, root1[None]], axis=0)
    xwf1 = _mm_call(x, wall1, D_HID)
    agg1 = _agg_kernel_hid(src, dst, edge_type, xwf1, scl, zer_hid)

    wall2 = jnp.concatenate([W2, root2[None]], axis=0)
    wall2p = jnp.pad(wall2, ((0, 0), (0, 0), (0, D_HID - D_OUT)))
    xwf2 = _mm2_fused(xwf1, agg1, b1.reshape(1, D_HID), wall2p)
    agg2 = _agg_kernel_hid(src, dst, edge_type, xwf2, scl, zer_hid)
    return _final_call(xwf2, agg2, b2.reshape(1, D_OUT))
